# baseline (speedup 1.0000x reference)
import functools

import jax
import jax.numpy as jnp
from jax import lax
from jax.experimental import pallas as pl
from jax.experimental.pallas import tpu as pltpu
from jax.experimental.pallas import tpu_sc as plsc

N_NODES = 10000
FEAT = 256
N_REL = 200
N_EDGES = 160000

NC, NS, L = 2, 16, 16
NTILES = NC * NS
E_PAD = 163840
EPT = E_PAD // NTILES
B = 64
NB = EPT // B
NCH = 4
CH = 64
NPAD = 10240
RPT = NPAD // NS
PKW = B * (4 + L)
NBT = E_PAD // B
REL_ROWS = 208
DUMMY = N_NODES


def _node_tables(node, bigw, wq):
    Bm = 400
    grid = N_NODES // Bm

    def body(nb_ref, bigw_ref, wq_ref, qp_ref, q3_ref, p3_ref, lw_ref, elw_ref):
        t = jnp.dot(nb_ref[...], bigw_ref[...], preferred_element_type=jnp.float32)
        p1 = t[:, 0:256]
        p3 = t[:, 256:512]
        q1 = jnp.dot(p1, wq_ref[...], preferred_element_type=jnp.float32)
        q3 = jnp.dot(p3, wq_ref[...], preferred_element_type=jnp.float32)
        p3_ref[...] = p3
        lw_ref[...] = t[:, 512:768]
        elw_ref[...] = t[:, 768:1024]
        q3_ref[...] = q3
        qp_ref[...] = jnp.concatenate(
            [x for k in range(NCH)
             for x in (q1[:, k * CH:(k + 1) * CH], p1[:, k * CH:(k + 1) * CH])],
            axis=1)

    out = pl.pallas_call(
        body,
        grid=(grid,),
        in_specs=[
            pl.BlockSpec((Bm, 256), lambda i: (i, 0)),
            pl.BlockSpec((256, 1024), lambda i: (0, 0)),
            pl.BlockSpec((256, 256), lambda i: (0, 0)),
        ],
        out_specs=[
            pl.BlockSpec((Bm, 512), lambda i: (i, 0)),
            pl.BlockSpec((Bm, 256), lambda i: (i, 0)),
            pl.BlockSpec((Bm, 256), lambda i: (i, 0)),
            pl.BlockSpec((Bm, 256), lambda i: (i, 0)),
            pl.BlockSpec((Bm, 256), lambda i: (i, 0)),
        ],
        out_shape=[
            jax.ShapeDtypeStruct((N_NODES, 512), jnp.float32),
            jax.ShapeDtypeStruct((N_NODES, 256), jnp.float32),
            jax.ShapeDtypeStruct((N_NODES, 256), jnp.float32),
            jax.ShapeDtypeStruct((N_NODES, 256), jnp.float32),
            jax.ShapeDtypeStruct((N_NODES, 256), jnp.float32),
        ],
    )(node, bigw, wq)
    return out


def _rel_tables(relp, w2, wq):
    def body(relp_ref, w2_ref, wq_ref, out_ref):
        p2 = jnp.dot(relp_ref[...], w2_ref[...], preferred_element_type=jnp.float32)
        q2 = jnp.dot(p2, wq_ref[...], preferred_element_type=jnp.float32)
        base = jnp.concatenate(
            [x for k in range(NCH)
             for x in (q2[:, k * CH:(k + 1) * CH], p2[:, k * CH:(k + 1) * CH])],
            axis=1)
        cs = jnp.sum(wq_ref[...], axis=0)
        zero = jnp.zeros((CH,), jnp.float32)
        cs_row = jnp.concatenate(
            [x for k in range(NCH) for x in (cs[k * CH:(k + 1) * CH], zero)])
        rows = lax.broadcasted_iota(jnp.int32, (REL_ROWS, 1), 0)
        out_ref[...] = jnp.where(rows == N_REL, cs_row[None, :], base)

    return pl.pallas_call(
        body,
        out_shape=jax.ShapeDtypeStruct((REL_ROWS, 512), jnp.float32),
    )(relp, w2, wq)


def _make_sc_edge():
    mesh = plsc.VectorSubcoreMesh(core_axis_name="c", subcore_axis_name="s")

    @functools.partial(
        pl.kernel,
        out_type=jax.ShapeDtypeStruct((NCH, NC, NPAD, 2 * CH), jnp.float32),
        mesh=mesh,
        scratch_types=[
            [pltpu.VMEM((PKW,), jnp.int32) for _ in range(2)],
            [pltpu.VMEM((B,), jnp.int32) for _ in range(2)],
            [pltpu.VMEM((B,), jnp.int32) for _ in range(2)],
            [pltpu.VMEM((B,), jnp.int32) for _ in range(2)],
            [pltpu.VMEM((B,), jnp.int32) for _ in range(2)],
            [pltpu.VMEM((B, 2 * CH), jnp.float32) for _ in range(2)],
            [pltpu.VMEM((B, CH), jnp.float32) for _ in range(2)],
            [pltpu.VMEM((B, 2 * CH), jnp.float32) for _ in range(2)],
            pltpu.VMEM((2 * CH,), jnp.float32),
            pltpu.VMEM((16, 2 * CH), jnp.float32),
            pltpu.VMEM_SHARED((NPAD, 2 * CH), jnp.float32),
            [[pltpu.SemaphoreType.DMA for _ in range(3)] for _ in range(2)],
            [pltpu.SemaphoreType.DMA for _ in range(2)],
            [pltpu.SemaphoreType.DMA for _ in range(2)],
        ],
        compiler_params=pltpu.CompilerParams(use_tc_tiling_on_sc=False),
    )
    def sc_edge(pk_h, qp_h, q3_h, rel_h, out_h,
                pbuf, isrc, idst, iet, iacc,
                qp_b, q3_b, rel_b, cs_b, zbuf, accum, sems, ssem, psem):
        cid = lax.axis_index("c")
        sid = lax.axis_index("s")
        wid = cid * NS + sid
        bbase = wid * NB

        def zb(i, carry):
            zbuf[i // 8, pl.ds((i % 8) * L, L)] = jnp.zeros((L,), jnp.float32)
            return carry

        lax.fori_loop(0, 16 * 8, zb, 0)
        rbase = sid * RPT

        def clear(i, carry):
            pltpu.sync_copy(zbuf, accum.at[pl.ds(rbase + i * 16, 16)])
            return carry

        lax.fori_loop(0, RPT // 16, clear, 0)
        plsc.subcore_barrier()

        def wait_scatter(p):
            pltpu.make_async_copy(qp_b[p], accum.at[iacc[p]], ssem[p]).wait()

        def fire_pk(j, p):
            pltpu.async_copy(pk_h.at[bbase + j], pbuf[p], psem[p])

        def issue(j, p, k):
            pltpu.make_async_copy(pk_h.at[bbase + j], pbuf[p], psem[p]).wait()
            for t in range(B // L):
                sl = pl.ds(t * L, L)
                isrc[p][sl] = pbuf[p][sl] + k
                idst[p][sl] = pbuf[p][pl.ds(B + t * L, L)] + k
                iet[p][sl] = pbuf[p][pl.ds(2 * B + t * L, L)] + k
                iacc[p][sl] = pbuf[p][pl.ds(3 * B + t * L, L)]
            pltpu.async_copy(qp_h.at[isrc[p]], qp_b[p], sems[p][0])
            pltpu.async_copy(q3_h.at[idst[p]], q3_b[p], sems[p][1])
            pltpu.async_copy(rel_h.at[iet[p]], rel_b[p], sems[p][2])

        for k in range(NCH):
            pltpu.sync_copy(rel_h.at[NCH * N_REL + k], cs_b)
            fire_pk(0, 0)
            issue(0, 0, k)
            fire_pk(1, 1)
            csv = [cs_b[pl.ds(t * L, L)] * jnp.float32(2.0 ** -24)
                   for t in range(CH // L)]

            def pair(i, carry):
                j = i * 2
                for p in range(2):
                    jb = j + p

                    @pl.when(jb + 1 < NB)
                    def _():
                        @pl.when(jb >= 1)
                        def _():
                            wait_scatter(1 - p)

                        issue(jb + 1, 1 - p, k)

                    pltpu.make_async_copy(qp_h.at[isrc[p]], qp_b[p],
                                          sems[p][0]).wait()
                    pltpu.make_async_copy(q3_h.at[idst[p]], q3_b[p],
                                          sems[p][1]).wait()
                    pltpu.make_async_copy(rel_h.at[iet[p]], rel_b[p],
                                          sems[p][2]).wait()

                    def edge1(e):
                        fs = pbuf[p][pl.ds(4 * B + e * L, L)].astype(
                            jnp.float32)
                        for t in range(CH // L):
                            sl = pl.ds(t * L, L)
                            sh = pl.ds(CH + t * L, L)
                            a = (qp_b[p][e, sl] + rel_b[p][e, sl]
                                 + q3_b[p][e, sl] + fs * csv[t])
                            a = jnp.where(a > 0, a, a * 0.01)
                            ex = jnp.exp(a)
                            pv = qp_b[p][e, sh] + rel_b[p][e, sh]
                            qp_b[p][e, sl] = ex
                            qp_b[p][e, sh] = ex * pv

                    plsc.parallel_loop(0, B, 1, unroll=8)(edge1)
                    pltpu.async_copy(qp_b[p], accum.at[iacc[p]], ssem[p],
                                     add=True)

                    @pl.when(jb + 2 < NB)
                    def _():
                        fire_pk(jb + 2, p)

                return carry

            lax.fori_loop(0, NB // 2, pair, 0)
            wait_scatter(0)
            wait_scatter(1)
            plsc.subcore_barrier()
            pltpu.sync_copy(accum.at[pl.ds(rbase, RPT)],
                            out_h.at[k, cid, pl.ds(rbase, RPT)])
            if k < NCH - 1:
                lax.fori_loop(0, RPT // 16, clear, 0)
            plsc.subcore_barrier()

    return sc_edge


_sc_edge = _make_sc_edge()


def _finalize(acc, p3, lw, elw, norm):
    Bm = 400
    grid = N_NODES // Bm

    def body(acc_ref, p3_ref, lw_ref, elw_ref, norm_ref, out_ref):
        acc = acc_ref[...]
        s = acc[:, 0, :, :] + acc[:, 1, :, :]
        den = jnp.concatenate([s[kk, :, 0:CH] for kk in range(NCH)], axis=1)
        num = jnp.concatenate([s[kk, :, CH:2 * CH] for kk in range(NCH)], axis=1)
        mask = den > 0
        he = jnp.where(mask, num / jnp.where(mask, den, 1.0) + p3_ref[...], 0.0)
        mask0 = den[:, 0:1] > 0
        out_ref[...] = (he * norm_ref[...]
                        + jnp.where(mask0, lw_ref[...], elw_ref[...]))

    return pl.pallas_call(
        body,
        grid=(grid,),
        in_specs=[
            pl.BlockSpec((NCH, NC, Bm, 2 * CH), lambda i: (0, 0, i, 0)),
            pl.BlockSpec((Bm, 256), lambda i: (i, 0)),
            pl.BlockSpec((Bm, 256), lambda i: (i, 0)),
            pl.BlockSpec((Bm, 256), lambda i: (i, 0)),
            pl.BlockSpec((Bm, 1), lambda i: (i, 0)),
        ],
        out_specs=pl.BlockSpec((Bm, 256), lambda i: (i, 0)),
        out_shape=jax.ShapeDtypeStruct((N_NODES, 256), jnp.float32),
    )(acc, p3, lw, elw, norm)


def kernel(node, rel, edge_index, edge_type, fre, norm,
           w_triplet, w_quad, loop_weight, evolve_loop_weight):
    w1 = w_triplet[0:256]
    w2 = w_triplet[256:512]
    w3 = w_triplet[512:768]
    bigw = jnp.concatenate([w1, w3, loop_weight, evolve_loop_weight], axis=1)
    qp, q3t, p3, lw, elw = _node_tables(node, bigw, w_quad)
    relp = jnp.concatenate([rel, jnp.zeros((REL_ROWS - N_REL, 256), jnp.float32)],
                           axis=0)
    qprel = _rel_tables(relp, w2, w_quad)

    qp_f = qp.reshape(N_NODES * NCH, 2 * CH)
    q3_f = q3t.reshape(N_NODES * NCH, CH)
    rel_f = qprel.reshape(REL_ROWS * NCH, 2 * CH)

    pad = E_PAD - N_EDGES
    srcp = jnp.concatenate([edge_index[0], jnp.zeros((pad,), jnp.int32)])
    dstp = jnp.concatenate([edge_index[1], jnp.full((pad,), DUMMY, jnp.int32)])
    etp = jnp.concatenate([edge_type, jnp.zeros((pad,), jnp.int32)])
    frep = jnp.concatenate([fre, jnp.zeros((pad,), jnp.float32)])
    freb = jnp.round(frep * jnp.float32(1 << 24)).astype(jnp.int32)
    packed = jnp.concatenate([
        (srcp * NCH).reshape(NBT, B),
        (jnp.minimum(dstp, N_NODES - 1) * NCH).reshape(NBT, B),
        (etp * NCH).reshape(NBT, B),
        dstp.reshape(NBT, B),
        jnp.broadcast_to(freb[:, None], (E_PAD, L)).reshape(NBT, B * L),
    ], axis=1)

    acc = _sc_edge(packed, qp_f, q3_f, rel_f)
    return _finalize(acc, p3, lw, elw, norm)


# speedup vs baseline: 1.0642x; 1.0642x over previous
import functools

import jax
import jax.numpy as jnp
from jax import lax
from jax.experimental import pallas as pl
from jax.experimental.pallas import tpu as pltpu
from jax.experimental.pallas import tpu_sc as plsc

N_NODES = 10000
FEAT = 256
N_REL = 200
N_EDGES = 160000

NC, NS, L = 2, 16, 16
NTILES = NC * NS
E_PAD = 163840
EPT = E_PAD // NTILES
B = 64
NB = EPT // B
NCH = 4
CH = 64
NPAD = 10240
RPT = NPAD // NS
PKW = B * (4 + L)
NBT = E_PAD // B
REL_ROWS = 208
DUMMY = N_NODES


def _node_tables(node, bigw, wq):
    Bm = 400
    grid = N_NODES // Bm

    def body(nb_ref, bigw_ref, wq_ref, qp_ref, q3_ref, p3_ref, lw_ref, elw_ref):
        t = jnp.dot(nb_ref[...], bigw_ref[...], preferred_element_type=jnp.float32)
        p1 = t[:, 0:256]
        p3 = t[:, 256:512]
        q1 = jnp.dot(p1, wq_ref[...], preferred_element_type=jnp.float32)
        q3 = jnp.dot(p3, wq_ref[...], preferred_element_type=jnp.float32)
        p3_ref[...] = p3
        lw_ref[...] = t[:, 512:768]
        elw_ref[...] = t[:, 768:1024]
        q3_ref[...] = q3
        qp_ref[...] = jnp.concatenate(
            [x for k in range(NCH)
             for x in (q1[:, k * CH:(k + 1) * CH], p1[:, k * CH:(k + 1) * CH])],
            axis=1)

    out = pl.pallas_call(
        body,
        grid=(grid,),
        in_specs=[
            pl.BlockSpec((Bm, 256), lambda i: (i, 0)),
            pl.BlockSpec((256, 1024), lambda i: (0, 0)),
            pl.BlockSpec((256, 256), lambda i: (0, 0)),
        ],
        out_specs=[
            pl.BlockSpec((Bm, 512), lambda i: (i, 0)),
            pl.BlockSpec((Bm, 256), lambda i: (i, 0)),
            pl.BlockSpec((Bm, 256), lambda i: (i, 0)),
            pl.BlockSpec((Bm, 256), lambda i: (i, 0)),
            pl.BlockSpec((Bm, 256), lambda i: (i, 0)),
        ],
        out_shape=[
            jax.ShapeDtypeStruct((N_NODES, 512), jnp.float32),
            jax.ShapeDtypeStruct((N_NODES, 256), jnp.float32),
            jax.ShapeDtypeStruct((N_NODES, 256), jnp.float32),
            jax.ShapeDtypeStruct((N_NODES, 256), jnp.float32),
            jax.ShapeDtypeStruct((N_NODES, 256), jnp.float32),
        ],
    )(node, bigw, wq)
    return out


def _rel_tables(relp, w2, wq):
    def body(relp_ref, w2_ref, wq_ref, out_ref):
        p2 = jnp.dot(relp_ref[...], w2_ref[...], preferred_element_type=jnp.float32)
        q2 = jnp.dot(p2, wq_ref[...], preferred_element_type=jnp.float32)
        base = jnp.concatenate(
            [x for k in range(NCH)
             for x in (q2[:, k * CH:(k + 1) * CH], p2[:, k * CH:(k + 1) * CH])],
            axis=1)
        cs = jnp.sum(wq_ref[...], axis=0)
        zero = jnp.zeros((CH,), jnp.float32)
        cs_row = jnp.concatenate(
            [x for k in range(NCH) for x in (cs[k * CH:(k + 1) * CH], zero)])
        rows = lax.broadcasted_iota(jnp.int32, (REL_ROWS, 1), 0)
        out_ref[...] = jnp.where(rows == N_REL, cs_row[None, :], base)

    return pl.pallas_call(
        body,
        out_shape=jax.ShapeDtypeStruct((REL_ROWS, 512), jnp.float32),
    )(relp, w2, wq)


def _make_sc_edge():
    mesh = plsc.VectorSubcoreMesh(core_axis_name="c", subcore_axis_name="s")

    @functools.partial(
        pl.kernel,
        out_type=jax.ShapeDtypeStruct((NCH, NC, NPAD, 2 * CH), jnp.float32),
        mesh=mesh,
        scratch_types=[
            [pltpu.VMEM((PKW,), jnp.int32) for _ in range(2)],
            [pltpu.VMEM((B,), jnp.int32) for _ in range(2)],
            [pltpu.VMEM((B,), jnp.int32) for _ in range(2)],
            [pltpu.VMEM((B,), jnp.int32) for _ in range(2)],
            [pltpu.VMEM((B,), jnp.int32) for _ in range(2)],
            [pltpu.VMEM((B, 2 * CH), jnp.float32) for _ in range(2)],
            [pltpu.VMEM((B, CH), jnp.float32) for _ in range(2)],
            [pltpu.VMEM((B, 2 * CH), jnp.float32) for _ in range(2)],
            pltpu.VMEM((2 * CH,), jnp.float32),
            pltpu.VMEM((16, 2 * CH), jnp.float32),
            pltpu.VMEM_SHARED((NPAD, 2 * CH), jnp.float32),
            [[pltpu.SemaphoreType.DMA for _ in range(3)] for _ in range(2)],
            [pltpu.SemaphoreType.DMA for _ in range(2)],
            [pltpu.SemaphoreType.DMA for _ in range(2)],
        ],
        compiler_params=pltpu.CompilerParams(use_tc_tiling_on_sc=False),
    )
    def sc_edge(pk_h, qp_h, q3_h, rel_h, out_h,
                pbuf, isrc, idst, iet, iacc,
                qp_b, q3_b, rel_b, cs_b, zbuf, accum, sems, ssem, psem):
        cid = lax.axis_index("c")
        sid = lax.axis_index("s")
        wid = cid * NS + sid
        bbase = wid * NB

        def zb(i, carry):
            zbuf[i // 8, pl.ds((i % 8) * L, L)] = jnp.zeros((L,), jnp.float32)
            return carry

        lax.fori_loop(0, 16 * 8, zb, 0)
        rbase = sid * RPT

        def clear(i, carry):
            pltpu.sync_copy(zbuf, accum.at[pl.ds(rbase + i * 16, 16)])
            return carry

        lax.fori_loop(0, RPT // 16, clear, 0)
        plsc.subcore_barrier()

        def wait_scatter(p):
            pltpu.make_async_copy(qp_b[p], accum.at[iacc[p]], ssem[p]).wait()

        def fire_pk(j, p):
            pltpu.async_copy(pk_h.at[bbase + j], pbuf[p], psem[p])

        def issue(j, p, k):
            pltpu.make_async_copy(pk_h.at[bbase + j], pbuf[p], psem[p]).wait()
            for t in range(B // L):
                sl = pl.ds(t * L, L)
                isrc[p][sl] = pbuf[p][sl] + k
                idst[p][sl] = pbuf[p][pl.ds(B + t * L, L)] + k
                iet[p][sl] = pbuf[p][pl.ds(2 * B + t * L, L)] + k
                iacc[p][sl] = pbuf[p][pl.ds(3 * B + t * L, L)]
            pltpu.async_copy(qp_h.at[isrc[p]], qp_b[p], sems[p][0])
            pltpu.async_copy(q3_h.at[idst[p]], q3_b[p], sems[p][1])
            pltpu.async_copy(rel_h.at[iet[p]], rel_b[p], sems[p][2])

        for k in range(NCH):
            pltpu.sync_copy(rel_h.at[NCH * N_REL + k], cs_b)
            fire_pk(0, 0)
            issue(0, 0, k)
            fire_pk(1, 1)
            csv = [cs_b[pl.ds(t * L, L)] * jnp.float32(2.0 ** -24)
                   for t in range(CH // L)]

            def pair(i, carry):
                j = i * 2
                for p in range(2):
                    jb = j + p

                    @pl.when(jb + 1 < NB)
                    def _():
                        @pl.when(jb >= 1)
                        def _():
                            wait_scatter(1 - p)

                        issue(jb + 1, 1 - p, k)

                    pltpu.make_async_copy(qp_h.at[isrc[p]], qp_b[p],
                                          sems[p][0]).wait()
                    pltpu.make_async_copy(q3_h.at[idst[p]], q3_b[p],
                                          sems[p][1]).wait()
                    pltpu.make_async_copy(rel_h.at[iet[p]], rel_b[p],
                                          sems[p][2]).wait()

                    def edge1(e):
                        fs = pbuf[p][pl.ds(4 * B + e * L, L)].astype(
                            jnp.float32)
                        for t in range(CH // L):
                            sl = pl.ds(t * L, L)
                            sh = pl.ds(CH + t * L, L)
                            a = (qp_b[p][e, sl] + rel_b[p][e, sl]
                                 + q3_b[p][e, sl] + fs * csv[t])
                            a = jnp.where(a > 0, a, a * 0.01)
                            ex = jnp.exp(a)
                            pv = qp_b[p][e, sh] + rel_b[p][e, sh]
                            qp_b[p][e, sl] = ex
                            qp_b[p][e, sh] = ex * pv

                    plsc.parallel_loop(0, B, 1, unroll=4)(edge1)
                    pltpu.async_copy(qp_b[p], accum.at[iacc[p]], ssem[p],
                                     add=True)

                    @pl.when(jb + 2 < NB)
                    def _():
                        fire_pk(jb + 2, p)

                return carry

            lax.fori_loop(0, NB // 2, pair, 0)
            wait_scatter(0)
            wait_scatter(1)
            plsc.subcore_barrier()
            pltpu.sync_copy(accum.at[pl.ds(rbase, RPT)],
                            out_h.at[k, cid, pl.ds(rbase, RPT)])
            if k < NCH - 1:
                lax.fori_loop(0, RPT // 16, clear, 0)
            plsc.subcore_barrier()

    return sc_edge


_sc_edge = _make_sc_edge()


def _finalize(acc, p3, lw, elw, norm):
    Bm = 400
    grid = N_NODES // Bm

    def body(acc_ref, p3_ref, lw_ref, elw_ref, norm_ref, out_ref):
        acc = acc_ref[...]
        s = acc[:, 0, :, :] + acc[:, 1, :, :]
        den = jnp.concatenate([s[kk, :, 0:CH] for kk in range(NCH)], axis=1)
        num = jnp.concatenate([s[kk, :, CH:2 * CH] for kk in range(NCH)], axis=1)
        mask = den > 0
        he = jnp.where(mask, num / jnp.where(mask, den, 1.0) + p3_ref[...], 0.0)
        mask0 = den[:, 0:1] > 0
        out_ref[...] = (he * norm_ref[...]
                        + jnp.where(mask0, lw_ref[...], elw_ref[...]))

    return pl.pallas_call(
        body,
        grid=(grid,),
        in_specs=[
            pl.BlockSpec((NCH, NC, Bm, 2 * CH), lambda i: (0, 0, i, 0)),
            pl.BlockSpec((Bm, 256), lambda i: (i, 0)),
            pl.BlockSpec((Bm, 256), lambda i: (i, 0)),
            pl.BlockSpec((Bm, 256), lambda i: (i, 0)),
            pl.BlockSpec((Bm, 1), lambda i: (i, 0)),
        ],
        out_specs=pl.BlockSpec((Bm, 256), lambda i: (i, 0)),
        out_shape=jax.ShapeDtypeStruct((N_NODES, 256), jnp.float32),
    )(acc, p3, lw, elw, norm)


def kernel(node, rel, edge_index, edge_type, fre, norm,
           w_triplet, w_quad, loop_weight, evolve_loop_weight):
    w1 = w_triplet[0:256]
    w2 = w_triplet[256:512]
    w3 = w_triplet[512:768]
    bigw = jnp.concatenate([w1, w3, loop_weight, evolve_loop_weight], axis=1)
    qp, q3t, p3, lw, elw = _node_tables(node, bigw, w_quad)
    relp = jnp.concatenate([rel, jnp.zeros((REL_ROWS - N_REL, 256), jnp.float32)],
                           axis=0)
    qprel = _rel_tables(relp, w2, w_quad)

    qp_f = qp.reshape(N_NODES * NCH, 2 * CH)
    q3_f = q3t.reshape(N_NODES * NCH, CH)
    rel_f = qprel.reshape(REL_ROWS * NCH, 2 * CH)

    pad = E_PAD - N_EDGES
    srcp = jnp.concatenate([edge_index[0], jnp.zeros((pad,), jnp.int32)])
    dstp = jnp.concatenate([edge_index[1], jnp.full((pad,), DUMMY, jnp.int32)])
    etp = jnp.concatenate([edge_type, jnp.zeros((pad,), jnp.int32)])
    frep = jnp.concatenate([fre, jnp.zeros((pad,), jnp.float32)])
    freb = jnp.round(frep * jnp.float32(1 << 24)).astype(jnp.int32)
    packed = jnp.concatenate([
        (srcp * NCH).reshape(NBT, B),
        (jnp.minimum(dstp, N_NODES - 1) * NCH).reshape(NBT, B),
        (etp * NCH).reshape(NBT, B),
        dstp.reshape(NBT, B),
        jnp.broadcast_to(freb[:, None], (E_PAD, L)).reshape(NBT, B * L),
    ], axis=1)

    acc = _sc_edge(packed, qp_f, q3_f, rel_f)
    return _finalize(acc, p3, lw, elw, norm)
